# Initial kernel scaffold; baseline (speedup 1.0000x reference)
import jax
import jax.numpy as jnp
from jax.experimental import pallas as pl


def kernel(x, edge_index, W0m, W0s, b0, W1m, W1s, b1, W2m, W2s, b2, Wp, bp, Wv, bv):
    raise NotImplementedError("write your pallas kernel here")


# speedup vs baseline: 3.2377x; 3.2377x over previous
import jax
import jax.numpy as jnp
from jax import lax
from jax.experimental import pallas as pl
from jax.experimental.pallas import tpu as pltpu
from jax.experimental.pallas import tpu_sc as plsc

N = 10000
E = 160000
D = 256
HALF = 128
NA = 128

_NC = 2
_NS = 16
_EPT = E // _NS
_CHUNK = 80
_NCHUNK = _EPT // _CHUNK
_RPT = 624
_TAIL = N - _RPT * _NS

_EPW = E // (_NC * _NS)
_DCHUNK = 40
_DNCH = _EPW // _DCHUNK


def _zero_fill(zbuf):
    def zfill(k, _):
        zbuf[k // 8, pl.ds((k % 8) * 16, 16)] = jnp.zeros((16,), jnp.float32)
        return 0
    lax.fori_loop(0, 16 * 8, zfill, 0)


def _zero_spmem(zbuf, sh_ref, s):
    r0 = s * _RPT

    def zacc(j, _):
        pltpu.sync_copy(zbuf, sh_ref.at[pl.ds(r0 + j * 16, 16)])
        return 0
    lax.fori_loop(0, _RPT // 16, zacc, 0)

    @pl.when(s == 0)
    def _():
        pltpu.sync_copy(zbuf, sh_ref.at[pl.ds(_RPT * _NS, _TAIL)])


def _writeback(sh_ref, out_hbm, s, row_off):
    r0 = s * _RPT
    pltpu.sync_copy(sh_ref.at[pl.ds(r0, _RPT)],
                    out_hbm.at[pl.ds(row_off + r0, _RPT)])

    @pl.when(s == 0)
    def _():
        pltpu.sync_copy(sh_ref.at[pl.ds(_RPT * _NS, _TAIL)],
                        out_hbm.at[pl.ds(row_off + _RPT * _NS, _TAIL)])


def _sc_agg_body(h_hbm, src_hbm, dst_hbm, agg_hbm,
                 acc_sh, src_v, dst_v, rows_v, zbuf, sem):
    c = lax.axis_index("c")
    s = lax.axis_index("s")

    _zero_fill(zbuf)
    _zero_spmem(zbuf, acc_sh, s)
    plsc.subcore_barrier()

    e0 = s * _EPT
    c_off = c * N

    def step(i, _):
        base = e0 + i * _CHUNK
        pltpu.sync_copy(src_hbm.at[pl.ds(base, _CHUNK)], src_v)
        pltpu.sync_copy(dst_hbm.at[pl.ds(base, _CHUNK)], dst_v)
        for j in range(_CHUNK // 16):
            src_v[pl.ds(j * 16, 16)] = src_v[pl.ds(j * 16, 16)] + c_off
        pltpu.async_copy(h_hbm.at[src_v], rows_v, sem).wait()
        pltpu.sync_copy(rows_v, acc_sh.at[dst_v], add=True)
        return 0
    lax.fori_loop(0, _NCHUNK, step, 0)

    plsc.subcore_barrier()
    _writeback(acc_sh, agg_hbm, s, c_off)


import functools


@functools.cache
def _get_sc_agg():
    mesh = plsc.VectorSubcoreMesh(core_axis_name="c", subcore_axis_name="s",
                                  num_cores=_NC, num_subcores=_NS)
    return pl.kernel(
        _sc_agg_body,
        out_type=jax.ShapeDtypeStruct((_NC * N, HALF), jnp.float32),
        mesh=mesh,
        scratch_types=[
            pltpu.VMEM_SHARED((N, HALF), jnp.float32),
            pltpu.VMEM((_CHUNK,), jnp.int32),
            pltpu.VMEM((_CHUNK,), jnp.int32),
            pltpu.VMEM((_CHUNK, HALF), jnp.float32),
            pltpu.VMEM((16, HALF), jnp.float32),
            pltpu.SemaphoreType.DMA,
        ],
        name="sc_segment_sum",
    )


def _sc_deg_body(dst_hbm, deg_hbm, deg_sh, dst_v, ones_v, zbuf, sem):
    c = lax.axis_index("c")
    s = lax.axis_index("s")

    _zero_fill(zbuf)

    def ofill(k, _):
        ones_v[k // 8, pl.ds((k % 8) * 16, 16)] = jnp.ones((16,), jnp.float32)
        return 0
    lax.fori_loop(0, _DCHUNK * 8, ofill, 0)

    _zero_spmem(zbuf, deg_sh, s)
    plsc.subcore_barrier()

    e0 = (c * _NS + s) * _EPW

    def step(i, _):
        base = e0 + i * _DCHUNK
        pltpu.sync_copy(dst_hbm.at[pl.ds(base, _DCHUNK)], dst_v)
        pltpu.sync_copy(ones_v, deg_sh.at[dst_v], add=True)
        return 0
    lax.fori_loop(0, _DNCH, step, 0)

    plsc.subcore_barrier()
    _writeback(deg_sh, deg_hbm, s, c * N)


@functools.cache
def _get_sc_deg():
    mesh = plsc.VectorSubcoreMesh(core_axis_name="c", subcore_axis_name="s",
                                  num_cores=_NC, num_subcores=_NS)
    return pl.kernel(
        _sc_deg_body,
        out_type=jax.ShapeDtypeStruct((_NC * N, HALF), jnp.float32),
        mesh=mesh,
        scratch_types=[
            pltpu.VMEM_SHARED((N, HALF), jnp.float32),
            pltpu.VMEM((_DCHUNK,), jnp.int32),
            pltpu.VMEM((_DCHUNK, HALF), jnp.float32),
            pltpu.VMEM((16, HALF), jnp.float32),
            pltpu.SemaphoreType.DMA,
        ],
        name="sc_degree",
    )

_RB = 1000
_NRB = N // _RB


def _tc_layer_body(a0, a1, dg0, dg1, h0, h1, wm, ws, bb, out):
    dinv = 1.0 / jnp.maximum(dg0[:, 0:1] + dg1[:, 0:1], 1.0)
    a = jnp.concatenate([a0[...], a1[...]], axis=1) * dinv
    h = jnp.concatenate([h0[...], h1[...]], axis=1)
    acc = jnp.dot(a, wm[...], preferred_element_type=jnp.float32)
    acc = acc + jnp.dot(h, ws[...], preferred_element_type=jnp.float32)
    out[...] = jnp.maximum(acc + bb[...], 0.0)


def _tc_layer(agg, deg, h, Wm, Ws, b):
    b2 = b.reshape(1, D)
    return pl.pallas_call(
        _tc_layer_body,
        grid=(2, _NRB),
        in_specs=[
            pl.BlockSpec((_RB, HALF), lambda cc, r: (r, 0)),
            pl.BlockSpec((_RB, HALF), lambda cc, r: (r + _NRB, 0)),
            pl.BlockSpec((_RB, HALF), lambda cc, r: (r, 0)),
            pl.BlockSpec((_RB, HALF), lambda cc, r: (r + _NRB, 0)),
            pl.BlockSpec((_RB, HALF), lambda cc, r: (r, 0)),
            pl.BlockSpec((_RB, HALF), lambda cc, r: (r + _NRB, 0)),
            pl.BlockSpec((D, HALF), lambda cc, r: (0, cc)),
            pl.BlockSpec((D, HALF), lambda cc, r: (0, cc)),
            pl.BlockSpec((1, HALF), lambda cc, r: (0, cc)),
        ],
        out_specs=pl.BlockSpec((_RB, HALF), lambda cc, r: (cc * _NRB + r, 0)),
        out_shape=jax.ShapeDtypeStruct((_NC * N, HALF), jnp.float32),
    )(agg, agg, deg, deg, h, h, Wm, Ws, b2)


def _tc_head_body(h0, h1, wp, bp, wv, bv, out_l, out_v):
    h = jnp.concatenate([h0[...], h1[...]], axis=1)
    out_l[...] = jnp.dot(h, wp[...], preferred_element_type=jnp.float32) + bp[...]
    out_v[...] = jnp.dot(h, wv[...], preferred_element_type=jnp.float32) + bv[...]


def _tc_head(h, Wp, bp, Wv, bv):
    return pl.pallas_call(
        _tc_head_body,
        grid=(_NRB,),
        in_specs=[
            pl.BlockSpec((_RB, HALF), lambda r: (r, 0)),
            pl.BlockSpec((_RB, HALF), lambda r: (r + _NRB, 0)),
            pl.BlockSpec((D, NA), lambda r: (0, 0)),
            pl.BlockSpec((1, NA), lambda r: (0, 0)),
            pl.BlockSpec((D, 1), lambda r: (0, 0)),
            pl.BlockSpec((1, 1), lambda r: (0, 0)),
        ],
        out_specs=[
            pl.BlockSpec((_RB, NA), lambda r: (r, 0)),
            pl.BlockSpec((_RB, 1), lambda r: (r, 0)),
        ],
        out_shape=[
            jax.ShapeDtypeStruct((N, NA), jnp.float32),
            jax.ShapeDtypeStruct((N, 1), jnp.float32),
        ],
    )(h, h, Wp, bp.reshape(1, NA), Wv, bv.reshape(1, 1))


def kernel(x, edge_index, W0m, W0s, b0, W1m, W1s, b1, W2m, W2s, b2,
           Wp, bp, Wv, bv):
    src = edge_index[0]
    dst = edge_index[1]
    h = x.reshape(N, _NC, HALF).transpose(1, 0, 2).reshape(_NC * N, HALF)

    sc_deg = _get_sc_deg()
    sc_agg = _get_sc_agg()
    deg = sc_deg(dst)
    agg = sc_agg(h, src, dst)
    h = _tc_layer(agg, deg, h, W0m, W0s, b0)
    agg = sc_agg(h, src, dst)
    h = _tc_layer(agg, deg, h, W1m, W1s, b1)
    agg = sc_agg(h, src, dst)
    h = _tc_layer(agg, deg, h, W2m, W2s, b2)

    logits, value = _tc_head(h, Wp, bp, Wv, bv)
    return (logits, value)
